# baseline (speedup 1.0000x reference)
import functools

import jax
import jax.numpy as jnp
from jax import lax
from jax.experimental import pallas as pl
from jax.experimental.pallas import tpu as pltpu
from jax.experimental.pallas import tpu_sc as plsc

SZ = 4096
TEMP = 10.0
EPS = 1e-20
LN2 = 0.6931471805599453
SQRT2 = 1.4142135623730951

NC = 2
NS = 16
NW = NC * NS
LANES = 16

R_SC = 32
ROW0 = SZ - R_SC
M_PER_W = R_SC // NW
RI = 16

_C0 = 1.4426991769054545
_C1 = -0.7212366511576747
_C2 = 0.4800737469155951
_C3 = -0.36592988270923904
_C4 = 0.31470880562262726
_C5 = -0.20438587444643186


def _log2(x):
    xi = plsc.bitcast(x, jnp.int32)
    e = (xi >> 23) - 127
    m = plsc.bitcast((xi & 0x007FFFFF) | 0x3F800000, jnp.float32)
    big = m >= SQRT2
    e = jnp.where(big, e + 1, e)
    m = jnp.where(big, m * 0.5, m)
    t = m - 1.0
    q = _C5
    q = q * t + _C4
    q = q * t + _C3
    q = q * t + _C2
    q = q * t + _C1
    q = q * t + _C0
    return e.astype(jnp.float32) + t * q


@functools.partial(
    pl.kernel,
    out_type=jax.ShapeDtypeStruct((R_SC * SZ,), jnp.float32),
    mesh=plsc.VectorSubcoreMesh(
        core_axis_name="c", subcore_axis_name="s", num_cores=NC, num_subcores=NS
    ),
    scratch_types=[
        pltpu.VMEM((32, 2, 128), jnp.float32),
        pltpu.VMEM((32, 2, 128), jnp.float32),
        pltpu.VMEM((SZ,), jnp.float32),
    ],
    compiler_params=pltpu.CompilerParams(needs_layout_passes=False),
)
def _gumbel_sc(gm_hbm, u_hbm, out_hbm, gm_v, u_v, o_v):
    wid = lax.axis_index("s") * NC + lax.axis_index("c")

    def row_body(r, _):
        row_local = wid * M_PER_W + r
        row = ROW0 + row_local
        pltpu.sync_copy(gm_hbm.at[row], gm_v)
        pltpu.sync_copy(u_hbm.at[row], u_v)

        def inner(j, _):
            tj = j >> 3
            l = (j & 7) * LANES
            ge = gm_v[tj, 0, pl.ds(l, LANES)]
            go = gm_v[tj, 1, pl.ds(l, LANES)]
            ue = u_v[tj, 0, pl.ds(l, LANES)]
            uo = u_v[tj, 1, pl.ds(l, LANES)]
            le = EPS - LN2 * _log2(ue + EPS)
            lo = EPS - LN2 * _log2(uo + EPS)
            darg = ((go - ge) - LN2 * _log2(lo / le)) * (1.0 / TEMP)
            o_v[pl.ds(j * LANES, LANES)] = 1.0 / (1.0 + jnp.exp(darg))
            return 0

        lax.fori_loop(0, SZ // LANES, inner, 0)
        pltpu.sync_copy(o_v, out_hbm.at[pl.ds(row_local * SZ, SZ)])
        return 0

    lax.fori_loop(0, M_PER_W, row_body, 0)


def _tc_body(gm_ref, u_ref, o_ref):
    ge = gm_ref[:, 0, :]
    go = gm_ref[:, 1, :]
    ue = u_ref[:, 0, :]
    uo = u_ref[:, 1, :]
    le = -jnp.log(ue + EPS) + EPS
    lo = -jnp.log(uo + EPS) + EPS
    darg = ((go - ge) + jnp.log(le / lo)) * (1.0 / TEMP)
    o_ref[...] = 1.0 / (1.0 + jnp.exp(darg))


def kernel(gen_matrix, u):
    gmt3 = gen_matrix.swapaxes(1, 2)
    ut3 = u.reshape(SZ, SZ, 2).swapaxes(1, 2)
    gmt4 = gen_matrix.reshape(SZ, 32, 128, 2).swapaxes(2, 3)
    ut4 = u.reshape(SZ, 32, 128, 2).swapaxes(2, 3)

    sc_out = _gumbel_sc(gmt4, ut4)

    tc_out = pl.pallas_call(
        _tc_body,
        grid=(ROW0 // RI,),
        in_specs=[
            pl.BlockSpec((RI, 2, SZ), lambda g: (g, 0, 0)),
            pl.BlockSpec((RI, 2, SZ), lambda g: (g, 0, 0)),
        ],
        out_specs=pl.BlockSpec((RI, SZ), lambda g: (g, 0)),
        out_shape=jax.ShapeDtypeStruct((SZ, SZ), jnp.float32),
    )(gmt3, ut3)

    return lax.dynamic_update_slice(tc_out, sc_out.reshape(R_SC, SZ), (ROW0, 0))


# speedup vs baseline: 1.6779x; 1.6779x over previous
import functools

import jax
import jax.numpy as jnp
from jax import lax
from jax.experimental import pallas as pl
from jax.experimental.pallas import tpu as pltpu
from jax.experimental.pallas import tpu_sc as plsc

SZ = 4096
TEMP = 10.0
EPS = 1e-20
LN2 = 0.6931471805599453
SQRT2 = 1.4142135623730951

NC = 2
NS = 16
NW = NC * NS
LANES = 16

R_SC = 64
ROW0 = SZ - R_SC
M_PER_W = R_SC // NW
RI = 64

_C0 = 1.4426991769054545
_C1 = -0.7212366511576747
_C2 = 0.4800737469155951
_C3 = -0.36592988270923904
_C4 = 0.31470880562262726
_C5 = -0.20438587444643186


def _log2(x):
    xi = plsc.bitcast(x, jnp.int32)
    e = (xi >> 23) - 127
    m = plsc.bitcast((xi & 0x007FFFFF) | 0x3F800000, jnp.float32)
    big = m >= SQRT2
    e = jnp.where(big, e + 1, e)
    m = jnp.where(big, m * 0.5, m)
    t = m - 1.0
    q = _C5
    q = q * t + _C4
    q = q * t + _C3
    q = q * t + _C2
    q = q * t + _C1
    q = q * t + _C0
    return e.astype(jnp.float32) + t * q


@functools.partial(
    pl.kernel,
    out_type=jax.ShapeDtypeStruct((R_SC * SZ,), jnp.float32),
    mesh=plsc.VectorSubcoreMesh(
        core_axis_name="c", subcore_axis_name="s", num_cores=NC, num_subcores=NS
    ),
    scratch_types=[
        pltpu.VMEM((32, 2, 128), jnp.float32),
        pltpu.VMEM((32, 2, 128), jnp.float32),
        pltpu.VMEM((SZ,), jnp.float32),
    ],
    compiler_params=pltpu.CompilerParams(needs_layout_passes=False),
)
def _gumbel_sc(gm_hbm, u_hbm, out_hbm, gm_v, u_v, o_v):
    wid = lax.axis_index("s") * NC + lax.axis_index("c")

    def row_body(r, _):
        row_local = wid * M_PER_W + r
        row = ROW0 + row_local
        pltpu.sync_copy(gm_hbm.at[row], gm_v)
        pltpu.sync_copy(u_hbm.at[row], u_v)

        def inner(j, _):
            tj = j >> 3
            l = (j & 7) * LANES
            ge = gm_v[tj, 0, pl.ds(l, LANES)]
            go = gm_v[tj, 1, pl.ds(l, LANES)]
            ue = u_v[tj, 0, pl.ds(l, LANES)]
            uo = u_v[tj, 1, pl.ds(l, LANES)]
            le = EPS - LN2 * _log2(ue + EPS)
            lo = EPS - LN2 * _log2(uo + EPS)
            darg = ((go - ge) - LN2 * _log2(lo / le)) * (1.0 / TEMP)
            o_v[pl.ds(j * LANES, LANES)] = 1.0 / (1.0 + jnp.exp(darg))
            return 0

        lax.fori_loop(0, SZ // LANES, inner, 0)
        pltpu.sync_copy(o_v, out_hbm.at[pl.ds(row_local * SZ, SZ)])
        return 0

    lax.fori_loop(0, M_PER_W, row_body, 0)


def _tc_body(gm_ref, u_ref, o_ref):
    ge = gm_ref[:, 0, :]
    go = gm_ref[:, 1, :]
    ue = u_ref[:, 0, :]
    uo = u_ref[:, 1, :]
    le = -jnp.log(ue + EPS) + EPS
    lo = -jnp.log(uo + EPS) + EPS
    darg = ((go - ge) + jnp.log(le / lo)) * (1.0 / TEMP)
    o_ref[...] = 1.0 / (1.0 + jnp.exp(darg))


def kernel(gen_matrix, u):
    gmt3 = gen_matrix.swapaxes(1, 2)
    ut3 = u.reshape(SZ, SZ, 2).swapaxes(1, 2)
    gmt4 = gen_matrix.reshape(SZ, 32, 128, 2).swapaxes(2, 3)
    ut4 = u.reshape(SZ, 32, 128, 2).swapaxes(2, 3)

    sc_out = _gumbel_sc(gmt4, ut4)

    tc_out = pl.pallas_call(
        _tc_body,
        grid=(ROW0 // RI,),
        in_specs=[
            pl.BlockSpec((RI, 2, SZ), lambda g: (g, 0, 0)),
            pl.BlockSpec((RI, 2, SZ), lambda g: (g, 0, 0)),
        ],
        out_specs=pl.BlockSpec((RI, SZ), lambda g: (g, 0)),
        out_shape=jax.ShapeDtypeStruct((SZ, SZ), jnp.float32),
    )(gmt3, ut3)

    return lax.dynamic_update_slice(tc_out, sc_out.reshape(R_SC, SZ), (ROW0, 0))


# speedup vs baseline: 1.8581x; 1.1074x over previous
import functools

import jax
import jax.numpy as jnp
from jax import lax
from jax.experimental import pallas as pl
from jax.experimental.pallas import tpu as pltpu
from jax.experimental.pallas import tpu_sc as plsc

SZ = 4096
TEMP = 10.0
EPS = 1e-20
LN2 = 0.6931471805599453
SQRT2 = 1.4142135623730951

NC = 2
NS = 16
NW = NC * NS
LANES = 16

R_SC = 128
ROW0 = SZ - R_SC
M_PER_W = R_SC // NW
RI = 128

_C0 = 1.4426991769054545
_C1 = -0.7212366511576747
_C2 = 0.4800737469155951
_C3 = -0.36592988270923904
_C4 = 0.31470880562262726
_C5 = -0.20438587444643186


def _log2(x):
    xi = plsc.bitcast(x, jnp.int32)
    e = (xi >> 23) - 127
    m = plsc.bitcast((xi & 0x007FFFFF) | 0x3F800000, jnp.float32)
    big = m >= SQRT2
    e = jnp.where(big, e + 1, e)
    m = jnp.where(big, m * 0.5, m)
    t = m - 1.0
    q = _C5
    q = q * t + _C4
    q = q * t + _C3
    q = q * t + _C2
    q = q * t + _C1
    q = q * t + _C0
    return e.astype(jnp.float32) + t * q


@functools.partial(
    pl.kernel,
    out_type=jax.ShapeDtypeStruct((R_SC * SZ,), jnp.float32),
    mesh=plsc.VectorSubcoreMesh(
        core_axis_name="c", subcore_axis_name="s", num_cores=NC, num_subcores=NS
    ),
    scratch_types=[
        pltpu.VMEM((32, 2, 128), jnp.float32),
        pltpu.VMEM((32, 2, 128), jnp.float32),
        pltpu.VMEM((SZ,), jnp.float32),
    ],
    compiler_params=pltpu.CompilerParams(needs_layout_passes=False),
)
def _gumbel_sc(gm_hbm, u_hbm, out_hbm, gm_v, u_v, o_v):
    wid = lax.axis_index("s") * NC + lax.axis_index("c")

    def row_body(r, _):
        row_local = wid * M_PER_W + r
        row = ROW0 + row_local
        pltpu.sync_copy(gm_hbm.at[row], gm_v)
        pltpu.sync_copy(u_hbm.at[row], u_v)

        def inner(j, _):
            tj = j >> 3
            l = (j & 7) * LANES
            ge = gm_v[tj, 0, pl.ds(l, LANES)]
            go = gm_v[tj, 1, pl.ds(l, LANES)]
            ue = u_v[tj, 0, pl.ds(l, LANES)]
            uo = u_v[tj, 1, pl.ds(l, LANES)]
            le = EPS - LN2 * _log2(ue + EPS)
            lo = EPS - LN2 * _log2(uo + EPS)
            darg = ((go - ge) - LN2 * _log2(lo / le)) * (1.0 / TEMP)
            o_v[pl.ds(j * LANES, LANES)] = 1.0 / (1.0 + jnp.exp(darg))
            return 0

        lax.fori_loop(0, SZ // LANES, inner, 0)
        pltpu.sync_copy(o_v, out_hbm.at[pl.ds(row_local * SZ, SZ)])
        return 0

    lax.fori_loop(0, M_PER_W, row_body, 0)


def _tc_body(gm_ref, u_ref, o_ref):
    ge = gm_ref[:, 0, :]
    go = gm_ref[:, 1, :]
    ue = u_ref[:, 0, :]
    uo = u_ref[:, 1, :]
    le = -jnp.log(ue + EPS) + EPS
    lo = -jnp.log(uo + EPS) + EPS
    darg = ((go - ge) + jnp.log(le / lo)) * (1.0 / TEMP)
    o_ref[...] = 1.0 / (1.0 + jnp.exp(darg))


def kernel(gen_matrix, u):
    gmt3 = gen_matrix.swapaxes(1, 2)
    ut3 = u.reshape(SZ, SZ, 2).swapaxes(1, 2)
    gmt4 = gen_matrix.reshape(SZ, 32, 128, 2).swapaxes(2, 3)
    ut4 = u.reshape(SZ, 32, 128, 2).swapaxes(2, 3)

    sc_out = _gumbel_sc(gmt4, ut4)

    tc_out = pl.pallas_call(
        _tc_body,
        grid=(ROW0 // RI,),
        in_specs=[
            pl.BlockSpec((RI, 2, SZ), lambda g: (g, 0, 0)),
            pl.BlockSpec((RI, 2, SZ), lambda g: (g, 0, 0)),
        ],
        out_specs=pl.BlockSpec((RI, SZ), lambda g: (g, 0)),
        out_shape=jax.ShapeDtypeStruct((SZ, SZ), jnp.float32),
    )(gmt3, ut3)

    return lax.dynamic_update_slice(tc_out, sc_out.reshape(R_SC, SZ), (ROW0, 0))


# speedup vs baseline: 1.9196x; 1.0331x over previous
import functools

import jax
import jax.numpy as jnp
from jax import lax
from jax.experimental import pallas as pl
from jax.experimental.pallas import tpu as pltpu
from jax.experimental.pallas import tpu_sc as plsc

SZ = 4096
TEMP = 10.0
EPS = 1e-20
LN2 = 0.6931471805599453
SQRT2 = 1.4142135623730951

NC = 2
NS = 16
NW = NC * NS
LANES = 16

R_SC = 64
ROW0 = SZ - R_SC
M_PER_W = R_SC // NW
RI = 192

_C0 = 1.4426991769054545
_C1 = -0.7212366511576747
_C2 = 0.4800737469155951
_C3 = -0.36592988270923904
_C4 = 0.31470880562262726
_C5 = -0.20438587444643186


def _log2(x):
    xi = plsc.bitcast(x, jnp.int32)
    e = (xi >> 23) - 127
    m = plsc.bitcast((xi & 0x007FFFFF) | 0x3F800000, jnp.float32)
    big = m >= SQRT2
    e = jnp.where(big, e + 1, e)
    m = jnp.where(big, m * 0.5, m)
    t = m - 1.0
    q = _C5
    q = q * t + _C4
    q = q * t + _C3
    q = q * t + _C2
    q = q * t + _C1
    q = q * t + _C0
    return e.astype(jnp.float32) + t * q


@functools.partial(
    pl.kernel,
    out_type=jax.ShapeDtypeStruct((R_SC * SZ,), jnp.float32),
    mesh=plsc.VectorSubcoreMesh(
        core_axis_name="c", subcore_axis_name="s", num_cores=NC, num_subcores=NS
    ),
    scratch_types=[
        pltpu.VMEM((32, 2, 128), jnp.float32),
        pltpu.VMEM((32, 2, 128), jnp.float32),
        pltpu.VMEM((SZ,), jnp.float32),
    ],
    compiler_params=pltpu.CompilerParams(needs_layout_passes=False),
)
def _gumbel_sc(gm_hbm, u_hbm, out_hbm, gm_v, u_v, o_v):
    wid = lax.axis_index("s") * NC + lax.axis_index("c")

    def row_body(r, _):
        row_local = wid * M_PER_W + r
        row = ROW0 + row_local
        pltpu.sync_copy(gm_hbm.at[row], gm_v)
        pltpu.sync_copy(u_hbm.at[row], u_v)

        def inner(j, _):
            tj = j >> 3
            l = (j & 7) * LANES
            ge = gm_v[tj, 0, pl.ds(l, LANES)]
            go = gm_v[tj, 1, pl.ds(l, LANES)]
            ue = u_v[tj, 0, pl.ds(l, LANES)]
            uo = u_v[tj, 1, pl.ds(l, LANES)]
            le = EPS - LN2 * _log2(ue + EPS)
            lo = EPS - LN2 * _log2(uo + EPS)
            darg = ((go - ge) - LN2 * _log2(lo / le)) * (1.0 / TEMP)
            o_v[pl.ds(j * LANES, LANES)] = 1.0 / (1.0 + jnp.exp(darg))
            return 0

        lax.fori_loop(0, SZ // LANES, inner, 0)
        pltpu.sync_copy(o_v, out_hbm.at[pl.ds(row_local * SZ, SZ)])
        return 0

    lax.fori_loop(0, M_PER_W, row_body, 0)


def _tc_body(gm_ref, u_ref, o_ref):
    ge = gm_ref[:, 0, :]
    go = gm_ref[:, 1, :]
    ue = u_ref[:, 0, :]
    uo = u_ref[:, 1, :]
    le = -jnp.log(ue + EPS) + EPS
    lo = -jnp.log(uo + EPS) + EPS
    darg = ((go - ge) + jnp.log(le / lo)) * (1.0 / TEMP)
    o_ref[...] = 1.0 / (1.0 + jnp.exp(darg))


def kernel(gen_matrix, u):
    gmt3 = gen_matrix.swapaxes(1, 2)
    ut3 = u.reshape(SZ, SZ, 2).swapaxes(1, 2)
    gmt4 = gen_matrix.reshape(SZ, 32, 128, 2).swapaxes(2, 3)
    ut4 = u.reshape(SZ, 32, 128, 2).swapaxes(2, 3)

    sc_out = _gumbel_sc(gmt4, ut4)

    tc_out = pl.pallas_call(
        _tc_body,
        grid=(ROW0 // RI,),
        in_specs=[
            pl.BlockSpec((RI, 2, SZ), lambda g: (g, 0, 0)),
            pl.BlockSpec((RI, 2, SZ), lambda g: (g, 0, 0)),
        ],
        out_specs=pl.BlockSpec((RI, SZ), lambda g: (g, 0)),
        out_shape=jax.ShapeDtypeStruct((SZ, SZ), jnp.float32),
    )(gmt3, ut3)

    return lax.dynamic_update_slice(tc_out, sc_out.reshape(R_SC, SZ), (ROW0, 0))


# speedup vs baseline: 1.9230x; 1.0018x over previous
import functools

import jax
import jax.numpy as jnp
from jax import lax
from jax.experimental import pallas as pl
from jax.experimental.pallas import tpu as pltpu
from jax.experimental.pallas import tpu_sc as plsc

SZ = 4096
TEMP = 10.0
EPS = 1e-20
LN2 = 0.6931471805599453
SQRT2 = 1.4142135623730951

NC = 2
NS = 16
NW = NC * NS
LANES = 16

R_SC = 64
ROW0 = SZ - R_SC
M_PER_W = R_SC // NW
RI = 224

_C0 = 1.4426991769054545
_C1 = -0.7212366511576747
_C2 = 0.4800737469155951
_C3 = -0.36592988270923904
_C4 = 0.31470880562262726
_C5 = -0.20438587444643186


def _log2(x):
    xi = plsc.bitcast(x, jnp.int32)
    e = (xi >> 23) - 127
    m = plsc.bitcast((xi & 0x007FFFFF) | 0x3F800000, jnp.float32)
    big = m >= SQRT2
    e = jnp.where(big, e + 1, e)
    m = jnp.where(big, m * 0.5, m)
    t = m - 1.0
    q = _C5
    q = q * t + _C4
    q = q * t + _C3
    q = q * t + _C2
    q = q * t + _C1
    q = q * t + _C0
    return e.astype(jnp.float32) + t * q


@functools.partial(
    pl.kernel,
    out_type=jax.ShapeDtypeStruct((R_SC * SZ,), jnp.float32),
    mesh=plsc.VectorSubcoreMesh(
        core_axis_name="c", subcore_axis_name="s", num_cores=NC, num_subcores=NS
    ),
    scratch_types=[
        pltpu.VMEM((32, 2, 128), jnp.float32),
        pltpu.VMEM((32, 2, 128), jnp.float32),
        pltpu.VMEM((SZ,), jnp.float32),
    ],
    compiler_params=pltpu.CompilerParams(needs_layout_passes=False),
)
def _gumbel_sc(gm_hbm, u_hbm, out_hbm, gm_v, u_v, o_v):
    wid = lax.axis_index("s") * NC + lax.axis_index("c")

    def row_body(r, _):
        row_local = wid * M_PER_W + r
        row = ROW0 + row_local
        pltpu.sync_copy(gm_hbm.at[row], gm_v)
        pltpu.sync_copy(u_hbm.at[row], u_v)

        def inner(j, _):
            tj = j >> 3
            l = (j & 7) * LANES
            ge = gm_v[tj, 0, pl.ds(l, LANES)]
            go = gm_v[tj, 1, pl.ds(l, LANES)]
            ue = u_v[tj, 0, pl.ds(l, LANES)]
            uo = u_v[tj, 1, pl.ds(l, LANES)]
            le = EPS - LN2 * _log2(ue + EPS)
            lo = EPS - LN2 * _log2(uo + EPS)
            darg = ((go - ge) - LN2 * _log2(lo / le)) * (1.0 / TEMP)
            o_v[pl.ds(j * LANES, LANES)] = 1.0 / (1.0 + jnp.exp(darg))
            return 0

        lax.fori_loop(0, SZ // LANES, inner, 0)
        pltpu.sync_copy(o_v, out_hbm.at[pl.ds(row_local * SZ, SZ)])
        return 0

    lax.fori_loop(0, M_PER_W, row_body, 0)


def _tc_body(gm_ref, u_ref, o_ref):
    ge = gm_ref[:, 0, :]
    go = gm_ref[:, 1, :]
    ue = u_ref[:, 0, :]
    uo = u_ref[:, 1, :]
    le = -jnp.log(ue + EPS) + EPS
    lo = -jnp.log(uo + EPS) + EPS
    darg = ((go - ge) + jnp.log(le / lo)) * (1.0 / TEMP)
    o_ref[...] = 1.0 / (1.0 + jnp.exp(darg))


def kernel(gen_matrix, u):
    gmt3 = gen_matrix.swapaxes(1, 2)
    ut3 = u.reshape(SZ, SZ, 2).swapaxes(1, 2)
    gmt4 = gen_matrix.reshape(SZ, 32, 128, 2).swapaxes(2, 3)
    ut4 = u.reshape(SZ, 32, 128, 2).swapaxes(2, 3)

    sc_out = _gumbel_sc(gmt4, ut4)

    tc_out = pl.pallas_call(
        _tc_body,
        grid=(ROW0 // RI,),
        in_specs=[
            pl.BlockSpec((RI, 2, SZ), lambda g: (g, 0, 0)),
            pl.BlockSpec((RI, 2, SZ), lambda g: (g, 0, 0)),
        ],
        out_specs=pl.BlockSpec((RI, SZ), lambda g: (g, 0)),
        out_shape=jax.ShapeDtypeStruct((SZ, SZ), jnp.float32),
    )(gmt3, ut3)

    return lax.dynamic_update_slice(tc_out, sc_out.reshape(R_SC, SZ), (ROW0, 0))


# speedup vs baseline: 1.9281x; 1.0027x over previous
import functools

import jax
import jax.numpy as jnp
from jax import lax
from jax.experimental import pallas as pl
from jax.experimental.pallas import tpu as pltpu
from jax.experimental.pallas import tpu_sc as plsc

SZ = 4096
TEMP = 10.0
EPS = 1e-20
LN2 = 0.6931471805599453
SQRT2 = 1.4142135623730951

NC = 2
NS = 16
NW = NC * NS
LANES = 16

R_SC = 64
ROW0 = SZ - R_SC
M_PER_W = R_SC // NW
RI = 288

_C0 = 1.4426991769054545
_C1 = -0.7212366511576747
_C2 = 0.4800737469155951
_C3 = -0.36592988270923904
_C4 = 0.31470880562262726
_C5 = -0.20438587444643186


def _log2(x):
    xi = plsc.bitcast(x, jnp.int32)
    e = (xi >> 23) - 127
    m = plsc.bitcast((xi & 0x007FFFFF) | 0x3F800000, jnp.float32)
    big = m >= SQRT2
    e = jnp.where(big, e + 1, e)
    m = jnp.where(big, m * 0.5, m)
    t = m - 1.0
    q = _C5
    q = q * t + _C4
    q = q * t + _C3
    q = q * t + _C2
    q = q * t + _C1
    q = q * t + _C0
    return e.astype(jnp.float32) + t * q


@functools.partial(
    pl.kernel,
    out_type=jax.ShapeDtypeStruct((R_SC * SZ,), jnp.float32),
    mesh=plsc.VectorSubcoreMesh(
        core_axis_name="c", subcore_axis_name="s", num_cores=NC, num_subcores=NS
    ),
    scratch_types=[
        pltpu.VMEM((32, 2, 128), jnp.float32),
        pltpu.VMEM((32, 2, 128), jnp.float32),
        pltpu.VMEM((SZ,), jnp.float32),
    ],
    compiler_params=pltpu.CompilerParams(needs_layout_passes=False),
)
def _gumbel_sc(gm_hbm, u_hbm, out_hbm, gm_v, u_v, o_v):
    wid = lax.axis_index("s") * NC + lax.axis_index("c")

    def row_body(r, _):
        row_local = wid * M_PER_W + r
        row = ROW0 + row_local
        pltpu.sync_copy(gm_hbm.at[row], gm_v)
        pltpu.sync_copy(u_hbm.at[row], u_v)

        def inner(j, _):
            tj = j >> 3
            l = (j & 7) * LANES
            ge = gm_v[tj, 0, pl.ds(l, LANES)]
            go = gm_v[tj, 1, pl.ds(l, LANES)]
            ue = u_v[tj, 0, pl.ds(l, LANES)]
            uo = u_v[tj, 1, pl.ds(l, LANES)]
            le = EPS - LN2 * _log2(ue + EPS)
            lo = EPS - LN2 * _log2(uo + EPS)
            darg = ((go - ge) - LN2 * _log2(lo / le)) * (1.0 / TEMP)
            o_v[pl.ds(j * LANES, LANES)] = 1.0 / (1.0 + jnp.exp(darg))
            return 0

        lax.fori_loop(0, SZ // LANES, inner, 0)
        pltpu.sync_copy(o_v, out_hbm.at[pl.ds(row_local * SZ, SZ)])
        return 0

    lax.fori_loop(0, M_PER_W, row_body, 0)


def _tc_body(gm_ref, u_ref, o_ref):
    ge = gm_ref[:, 0, :]
    go = gm_ref[:, 1, :]
    ue = u_ref[:, 0, :]
    uo = u_ref[:, 1, :]
    le = -jnp.log(ue + EPS) + EPS
    lo = -jnp.log(uo + EPS) + EPS
    darg = ((go - ge) + jnp.log(le / lo)) * (1.0 / TEMP)
    o_ref[...] = 1.0 / (1.0 + jnp.exp(darg))


def kernel(gen_matrix, u):
    gmt3 = gen_matrix.swapaxes(1, 2)
    ut3 = u.reshape(SZ, SZ, 2).swapaxes(1, 2)
    gmt4 = gen_matrix.reshape(SZ, 32, 128, 2).swapaxes(2, 3)
    ut4 = u.reshape(SZ, 32, 128, 2).swapaxes(2, 3)

    sc_out = _gumbel_sc(gmt4, ut4)

    tc_out = pl.pallas_call(
        _tc_body,
        grid=(ROW0 // RI,),
        in_specs=[
            pl.BlockSpec((RI, 2, SZ), lambda g: (g, 0, 0)),
            pl.BlockSpec((RI, 2, SZ), lambda g: (g, 0, 0)),
        ],
        out_specs=pl.BlockSpec((RI, SZ), lambda g: (g, 0)),
        out_shape=jax.ShapeDtypeStruct((SZ, SZ), jnp.float32),
    )(gmt3, ut3)

    return lax.dynamic_update_slice(tc_out, sc_out.reshape(R_SC, SZ), (ROW0, 0))
